# baseline (speedup 1.0000x reference)
import functools

import jax
import jax.numpy as jnp
from jax import lax
from jax.experimental import pallas as pl
from jax.experimental.pallas import tpu as pltpu
from jax.experimental.pallas import tpu_sc as plsc

_NUM_TILES = 32
_GROUP = 128
_NS = 50
_NB = 16384
_B = _NB * _NS
_D = 32
_NBB = _NB // (_GROUP * _NUM_TILES)
_GPT = _NBB * _NS
_NSLOT = 4
_NSTEP = _GPT // _NSLOT
_V = 1000000

_mesh = plsc.VectorSubcoreMesh(core_axis_name="c", subcore_axis_name="s")


@functools.partial(
    pl.kernel,
    mesh=_mesh,
    compiler_params=pltpu.CompilerParams(use_tc_tiling_on_sc=False),
    out_type=jax.ShapeDtypeStruct((_B, _D), jnp.float32),
    scratch_types=(
        [pltpu.VMEM((_NS, _NBB * _GROUP), jnp.int32),
         pltpu.VMEM((2 * _NSLOT, _GROUP, _D), jnp.float32)]
        + [pltpu.SemaphoreType.DMA] * (4 * _NSLOT)
    ),
)
def _gather_kernel(idx_hbm, table_hbm, out_hbm, idx_v, rows_v, *sems):
    gsems = sems[:2 * _NSLOT]
    osems = sems[2 * _NSLOT:]
    wid = lax.axis_index("s") * 2 + lax.axis_index("c")
    pltpu.sync_copy(idx_hbm.at[:, pl.ds(wid * _NBB * _GROUP, _NBB * _GROUP)],
                    idx_v)

    def _gather(g, b):
        j = g // _NS
        s = g % _NS
        return pltpu.make_async_copy(
            table_hbm.at[idx_v.at[s, pl.ds(j * _GROUP, _GROUP)]],
            rows_v.at[b], gsems[b])

    def _out(g, b):
        j = g // _NS
        s = g % _NS
        row0 = s * _NB + (wid * _NBB + j) * _GROUP
        return pltpu.make_async_copy(
            rows_v.at[b], out_hbm.at[pl.ds(row0, _GROUP)], osems[b])

    def _start_gathers(st, h):
        for j in range(_NSLOT):
            _gather(st * _NSLOT + j, h * _NSLOT + j).start()

    def _drain_gathers_start_outs(st, h):
        for j in range(_NSLOT):
            b = h * _NSLOT + j
            _gather(st * _NSLOT + j, b).wait()
            _out(st * _NSLOT + j, b).start()

    def _wait_outs(st, h):
        for j in range(_NSLOT):
            _out(st * _NSLOT + j, h * _NSLOT + j).wait()

    _start_gathers(0, 0)
    _start_gathers(1, 1)
    _drain_gathers_start_outs(0, 0)

    @pl.loop(0, (_NSTEP - 2) // 2)
    def _steps(it):
        sa = 2 + 2 * it
        _wait_outs(sa - 2, 0)
        _start_gathers(sa, 0)
        _drain_gathers_start_outs(sa - 1, 1)
        sb = sa + 1
        _wait_outs(sb - 2, 1)
        _start_gathers(sb, 1)
        _drain_gathers_start_outs(sb - 1, 0)

    _drain_gathers_start_outs(_NSTEP - 1, 1)
    _wait_outs(_NSTEP - 2, 0)
    _wait_outs(_NSTEP - 1, 1)


_CBLK = 2048


def _tc_pack_body(x_ref, y_ref):
    y_ref[:, 0:_D] = x_ref[...].T


def _tc_pack(wt):
    return pl.pallas_call(
        _tc_pack_body,
        grid=((_V + _CBLK - 1) // _CBLK,),
        in_specs=[pl.BlockSpec((_D, _CBLK), lambda i: (0, i))],
        out_specs=pl.BlockSpec((_CBLK, 128), lambda i: (i, 0)),
        out_shape=jax.ShapeDtypeStruct((_V, 128), jnp.float32),
    )(wt)


def _tc_out_body(x_ref, y_ref):
    t = x_ref[0, :, :].T
    for rt in range(4):
        y_ref[0, rt, 0, :, :] = t[8 * rt:8 * rt + 8, :]


def _tc_out(x3):
    return pl.pallas_call(
        _tc_out_body,
        grid=(_NS, _NB // _GROUP),
        in_specs=[pl.BlockSpec((1, _GROUP, _D), lambda s, ct: (s, ct, 0))],
        out_specs=pl.BlockSpec((1, 4, 1, 8, 128),
                               lambda s, ct: (s, 0, ct, 0, 0)),
        out_shape=jax.ShapeDtypeStruct((_NS, 4, 128, 8, 128), jnp.float32),
    )(x3)


def kernel(input, weight):
    idx_t = input.T.astype(jnp.int32) * 4
    table = _tc_pack(weight.T).reshape(_V * 4, _D)
    out2s = _gather_kernel(idx_t, table)
    o5 = _tc_out(out2s.reshape(_NS, _NB, _D))
    return o5.transpose(2, 4, 0, 1, 3).reshape(_NB, _NS, _D)


# speedup vs baseline: 3.4023x; 3.4023x over previous
import functools

import jax
import jax.numpy as jnp
from jax import lax
from jax.experimental import pallas as pl
from jax.experimental.pallas import tpu as pltpu
from jax.experimental.pallas import tpu_sc as plsc

_NUM_TILES = 32
_GROUP = 128
_NS = 50
_NB = 16384
_B = _NB * _NS
_D = 32
_NBB = _NB // (_GROUP * _NUM_TILES)
_GPT = _NBB * _NS
_NSLOT = 2
_NSTEP = _GPT // _NSLOT
_V = 1000000

_mesh = plsc.VectorSubcoreMesh(core_axis_name="c", subcore_axis_name="s")


@functools.partial(
    pl.kernel,
    mesh=_mesh,
    compiler_params=pltpu.CompilerParams(use_tc_tiling_on_sc=False,
                                         needs_layout_passes=False),
    out_type=jax.ShapeDtypeStruct((_NS, 4, 128, 8, 128), jnp.float32),
    scratch_types=(
        [pltpu.VMEM((_NS, _NBB * _GROUP), jnp.int32),
         pltpu.VMEM((2 * _NSLOT, _GROUP, _D), jnp.float32),
         pltpu.VMEM((2 * _NSLOT, 4, 8, 128), jnp.float32)]
        + [pltpu.SemaphoreType.DMA] * (4 * _NSLOT)
    ),
)
def _gather_kernel(idx_hbm, table_hbm, o5_hbm, idx_v, rows_v, tr_v, *sems):
    gsems = sems[:2 * _NSLOT]
    osems = sems[2 * _NSLOT:]
    wid = lax.axis_index("s") * 2 + lax.axis_index("c")
    pltpu.sync_copy(idx_hbm.at[:, pl.ds(wid * _NBB * _GROUP, _NBB * _GROUP)],
                    idx_v)
    iota = lax.iota(jnp.int32, 16)
    rowv = [iota + 16 * c for c in range(_GROUP // 16)]

    def _gather(g, b):
        j = g // _NS
        s = g % _NS
        return pltpu.make_async_copy(
            table_hbm.at[idx_v.at[s, pl.ds(j * _GROUP, _GROUP)]],
            rows_v.at[b], gsems[b])

    def _transpose(b):
        for d in range(_D):
            dv = jnp.full((16,), d, jnp.int32)
            for c in range(_GROUP // 16):
                vals = plsc.load_gather(rows_v.at[b], [rowv[c], dv])
                tr_v[b, d // 8, d % 8, pl.ds(16 * c, 16)] = vals

    def _out(g, b):
        j = g // _NS
        s = g % _NS
        ct = wid * _NBB + j
        return pltpu.make_async_copy(tr_v.at[b], o5_hbm.at[s, :, ct],
                                     osems[b])

    def _start_gathers(st, h):
        for j in range(_NSLOT):
            _gather(st * _NSLOT + j, h * _NSLOT + j).start()

    def _drain_gathers_start_outs(st, h):
        for j in range(_NSLOT):
            b = h * _NSLOT + j
            _gather(st * _NSLOT + j, b).wait()
            _transpose(b)
            _out(st * _NSLOT + j, b).start()

    def _wait_outs(st, h):
        for j in range(_NSLOT):
            _out(st * _NSLOT + j, h * _NSLOT + j).wait()

    _start_gathers(0, 0)
    _start_gathers(1, 1)
    _drain_gathers_start_outs(0, 0)

    @pl.loop(0, (_NSTEP - 2) // 2)
    def _steps(it):
        sa = 2 + 2 * it
        _wait_outs(sa - 2, 0)
        _start_gathers(sa, 0)
        _drain_gathers_start_outs(sa - 1, 1)
        sb = sa + 1
        _wait_outs(sb - 2, 1)
        _start_gathers(sb, 1)
        _drain_gathers_start_outs(sb - 1, 0)

    _drain_gathers_start_outs(_NSTEP - 1, 1)
    _wait_outs(_NSTEP - 2, 0)
    _wait_outs(_NSTEP - 1, 1)


_CBLK = 2048


def _tc_pack_body(x_ref, y_ref):
    y_ref[:, 0:_D] = x_ref[...].T


def _tc_pack(wt):
    return pl.pallas_call(
        _tc_pack_body,
        grid=((_V + _CBLK - 1) // _CBLK,),
        in_specs=[pl.BlockSpec((_D, _CBLK), lambda i: (0, i))],
        out_specs=pl.BlockSpec((_CBLK, 128), lambda i: (i, 0)),
        out_shape=jax.ShapeDtypeStruct((_V, 128), jnp.float32),
    )(wt)


def kernel(input, weight):
    idx_t = input.T.astype(jnp.int32) * 4
    table = _tc_pack(weight.T).reshape(_V * 4, _D)
    o5 = _gather_kernel(idx_t, table)
    return o5.transpose(2, 4, 0, 1, 3).reshape(_NB, _NS, _D)


# speedup vs baseline: 5.4736x; 1.6088x over previous
import functools

import jax
import jax.numpy as jnp
from jax import lax
from jax.experimental import pallas as pl
from jax.experimental.pallas import tpu as pltpu
from jax.experimental.pallas import tpu_sc as plsc

_NUM_TILES = 32
_GROUP = 128
_NS = 50
_NB = 16384
_B = _NB * _NS
_D = 32
_NBB = _NB // (_GROUP * _NUM_TILES)
_GPT = _NBB * _NS
_NSLOT = 2
_NSTEP = _GPT // _NSLOT
_V = 1000000

_mesh = plsc.VectorSubcoreMesh(core_axis_name="c", subcore_axis_name="s")


@functools.partial(
    pl.kernel,
    mesh=_mesh,
    compiler_params=pltpu.CompilerParams(use_tc_tiling_on_sc=False,
                                         needs_layout_passes=False),
    out_type=jax.ShapeDtypeStruct((_NS, 4, 128, 8, 128), jnp.float32),
    scratch_types=(
        [pltpu.VMEM((_NS, _NBB * _GROUP), jnp.int32),
         pltpu.VMEM((2 * _NSLOT, _GROUP, _D), jnp.float32),
         pltpu.VMEM((2 * _NSLOT, 4, 8, 129), jnp.float32)]
        + [pltpu.SemaphoreType.DMA] * (4 * _NSLOT)
    ),
)
def _gather_kernel(idx_hbm, table_hbm, o5_hbm, idx_v, rows_v, tr_v, *sems):
    gsems = sems[:2 * _NSLOT]
    osems = sems[2 * _NSLOT:]
    wid = lax.axis_index("s") * 2 + lax.axis_index("c")
    pltpu.sync_copy(idx_hbm.at[:, pl.ds(wid * _NBB * _GROUP, _NBB * _GROUP)],
                    idx_v)
    iota = lax.iota(jnp.int32, 16)
    rtv = [(iota + 16 * c) // 8 for c in range(2)]
    rrv = [(iota + 16 * c) % 8 for c in range(2)]

    def _gather(g, b):
        j = g // _NS
        s = g % _NS
        return pltpu.make_async_copy(
            table_hbm.at[idx_v.at[s, pl.ds(j * _GROUP, _GROUP)]],
            rows_v.at[b], gsems[b])

    def _transpose(b):
        for l in range(_GROUP):
            lv = jnp.full((16,), l, jnp.int32)
            for c in range(2):
                vals = rows_v[b, l, pl.ds(16 * c, 16)]
                plsc.store_scatter(tr_v.at[b], [rtv[c], rrv[c], lv], vals)

    def _out(g, b):
        j = g // _NS
        s = g % _NS
        ct = wid * _NBB + j
        return pltpu.make_async_copy(tr_v.at[b, :, :, pl.ds(0, 128)],
                                     o5_hbm.at[s, :, ct], osems[b])

    def _start_gathers(st, h):
        for j in range(_NSLOT):
            _gather(st * _NSLOT + j, h * _NSLOT + j).start()

    def _drain_gathers_start_outs(st, h):
        for j in range(_NSLOT):
            b = h * _NSLOT + j
            _gather(st * _NSLOT + j, b).wait()
            _transpose(b)
            _out(st * _NSLOT + j, b).start()

    def _wait_outs(st, h):
        for j in range(_NSLOT):
            _out(st * _NSLOT + j, h * _NSLOT + j).wait()

    _start_gathers(0, 0)
    _start_gathers(1, 1)
    _drain_gathers_start_outs(0, 0)

    @pl.loop(0, (_NSTEP - 2) // 2)
    def _steps(it):
        sa = 2 + 2 * it
        _wait_outs(sa - 2, 0)
        _start_gathers(sa, 0)
        _drain_gathers_start_outs(sa - 1, 1)
        sb = sa + 1
        _wait_outs(sb - 2, 1)
        _start_gathers(sb, 1)
        _drain_gathers_start_outs(sb - 1, 0)

    _drain_gathers_start_outs(_NSTEP - 1, 1)
    _wait_outs(_NSTEP - 2, 0)
    _wait_outs(_NSTEP - 1, 1)


_CBLK = 2048


def _tc_pack_body(x_ref, y_ref):
    y_ref[:, 0:_D] = x_ref[...].T


def _tc_pack(wt):
    return pl.pallas_call(
        _tc_pack_body,
        grid=((_V + _CBLK - 1) // _CBLK,),
        in_specs=[pl.BlockSpec((_D, _CBLK), lambda i: (0, i))],
        out_specs=pl.BlockSpec((_CBLK, 128), lambda i: (i, 0)),
        out_shape=jax.ShapeDtypeStruct((_V, 128), jnp.float32),
    )(wt)


def kernel(input, weight):
    idx_t = input.T.astype(jnp.int32) * 4
    table = _tc_pack(weight.T).reshape(_V * 4, _D)
    o5 = _gather_kernel(idx_t, table)
    return o5.transpose(2, 4, 0, 1, 3).reshape(_NB, _NS, _D)
